# Initial kernel scaffold; baseline (speedup 1.0000x reference)
import jax
import jax.numpy as jnp
from jax.experimental import pallas as pl


def kernel(x, gate_weight):
    raise NotImplementedError("write your pallas kernel here")


# speedup vs baseline: 2.0465x; 2.0465x over previous
import functools

import jax
import jax.numpy as jnp
from jax.experimental import pallas as pl
from jax.experimental.pallas import tpu as pltpu

HIDDEN_SIZE = 768
NUM_EXPERTS = 64
N_TOKENS = 32768
BLOCK_T = 1024


def _gate_block(x_ref, wt_ref, gates_ref, idx_ref):
    logits = jnp.dot(x_ref[...], wt_ref[...], preferred_element_type=jnp.float32)
    col = jax.lax.broadcasted_iota(jnp.int32, logits.shape, 1)
    neg_big = jnp.finfo(jnp.float32).min

    m1 = jnp.max(logits, axis=-1)
    i1 = jnp.min(jnp.where(logits == m1[:, None], col, NUM_EXPERTS), axis=-1)
    masked = jnp.where(col == i1[:, None], neg_big, logits)
    m2 = jnp.max(masked, axis=-1)
    i2 = jnp.min(jnp.where(masked == m2[:, None], col, NUM_EXPERTS), axis=-1)

    e = jnp.exp(m2 - m1)
    g1 = 1.0 / (1.0 + e)
    g2 = 1.0 - g1
    gates_ref[...] = jnp.stack([g1, g2], axis=-1)
    idx_ref[...] = jnp.stack([i1, i2], axis=-1)


@jax.jit
def kernel(x, gate_weight):
    wt = gate_weight.T
    n = x.shape[0]
    grid = (n // BLOCK_T,)
    gates, idx = pl.pallas_call(
        _gate_block,
        grid=grid,
        in_specs=[
            pl.BlockSpec((BLOCK_T, HIDDEN_SIZE), lambda i: (i, 0)),
            pl.BlockSpec((HIDDEN_SIZE, NUM_EXPERTS), lambda i: (0, 0)),
        ],
        out_specs=[
            pl.BlockSpec((BLOCK_T, 2), lambda i: (i, 0)),
            pl.BlockSpec((BLOCK_T, 2), lambda i: (i, 0)),
        ],
        out_shape=[
            jax.ShapeDtypeStruct((n, 2), jnp.float32),
            jax.ShapeDtypeStruct((n, 2), jnp.int32),
        ],
    )(x, wt)
    return (gates, idx)


# speedup vs baseline: 2.3385x; 1.1427x over previous
import functools

import jax
import jax.numpy as jnp
from jax.experimental import pallas as pl
from jax.experimental.pallas import tpu as pltpu

HIDDEN_SIZE = 768
NUM_EXPERTS = 64
N_TOKENS = 32768
BLOCK_T = 2048


def _gate_block(x_ref, wt_ref, gates_ref, idx_ref):
    logits = jnp.dot(x_ref[...], wt_ref[...], preferred_element_type=jnp.float32)
    col = jax.lax.broadcasted_iota(jnp.int32, logits.shape, 1)
    neg_big = jnp.finfo(jnp.float32).min

    m1 = jnp.max(logits, axis=-1)
    i1 = jnp.min(jnp.where(logits == m1[:, None], col, NUM_EXPERTS), axis=-1)
    masked = jnp.where(col == i1[:, None], neg_big, logits)
    m2 = jnp.max(masked, axis=-1)
    i2 = jnp.min(jnp.where(masked == m2[:, None], col, NUM_EXPERTS), axis=-1)

    e = jnp.exp(m2 - m1)
    g1 = 1.0 / (1.0 + e)
    g2 = 1.0 - g1
    gates_ref[...] = jnp.stack([g1, g2], axis=-1)
    idx_ref[...] = jnp.stack([i1, i2], axis=-1)


@jax.jit
def kernel(x, gate_weight):
    wt = gate_weight.T
    n = x.shape[0]
    grid = (n // BLOCK_T,)
    gates, idx = pl.pallas_call(
        _gate_block,
        grid=grid,
        in_specs=[
            pl.BlockSpec((BLOCK_T, HIDDEN_SIZE), lambda i: (i, 0)),
            pl.BlockSpec((HIDDEN_SIZE, NUM_EXPERTS), lambda i: (0, 0)),
        ],
        out_specs=[
            pl.BlockSpec((BLOCK_T, 2), lambda i: (i, 0)),
            pl.BlockSpec((BLOCK_T, 2), lambda i: (i, 0)),
        ],
        out_shape=[
            jax.ShapeDtypeStruct((n, 2), jnp.float32),
            jax.ShapeDtypeStruct((n, 2), jnp.int32),
        ],
    )(x, wt)
    return (gates, idx)


# speedup vs baseline: 2.5096x; 1.0732x over previous
import functools

import jax
import jax.numpy as jnp
from jax.experimental import pallas as pl
from jax.experimental.pallas import tpu as pltpu

HIDDEN_SIZE = 768
NUM_EXPERTS = 64
N_TOKENS = 32768
BLOCK_T = 4096


def _gate_block(x_ref, wt_ref, gates_ref, idx_ref):
    logits = jnp.dot(x_ref[...], wt_ref[...], preferred_element_type=jnp.float32)
    col = jax.lax.broadcasted_iota(jnp.int32, logits.shape, 1)
    neg_big = jnp.finfo(jnp.float32).min

    m1 = jnp.max(logits, axis=-1)
    i1 = jnp.min(jnp.where(logits == m1[:, None], col, NUM_EXPERTS), axis=-1)
    masked = jnp.where(col == i1[:, None], neg_big, logits)
    m2 = jnp.max(masked, axis=-1)
    i2 = jnp.min(jnp.where(masked == m2[:, None], col, NUM_EXPERTS), axis=-1)

    e = jnp.exp(m2 - m1)
    g1 = 1.0 / (1.0 + e)
    g2 = 1.0 - g1
    gates_ref[...] = jnp.stack([g1, g2], axis=-1)
    idx_ref[...] = jnp.stack([i1, i2], axis=-1)


@jax.jit
def kernel(x, gate_weight):
    wt = gate_weight.T
    n = x.shape[0]
    grid = (n // BLOCK_T,)
    gates, idx = pl.pallas_call(
        _gate_block,
        grid=grid,
        in_specs=[
            pl.BlockSpec((BLOCK_T, HIDDEN_SIZE), lambda i: (i, 0)),
            pl.BlockSpec((HIDDEN_SIZE, NUM_EXPERTS), lambda i: (0, 0)),
        ],
        out_specs=[
            pl.BlockSpec((BLOCK_T, 2), lambda i: (i, 0)),
            pl.BlockSpec((BLOCK_T, 2), lambda i: (i, 0)),
        ],
        out_shape=[
            jax.ShapeDtypeStruct((n, 2), jnp.float32),
            jax.ShapeDtypeStruct((n, 2), jnp.int32),
        ],
    )(x, wt)
    return (gates, idx)


# speedup vs baseline: 5.0724x; 2.0212x over previous
import jax
import jax.numpy as jnp
from jax import lax
from jax.experimental import pallas as pl

HIDDEN_SIZE = 768
NUM_EXPERTS = 64
N_TOKENS = 32768
BLOCK_T = 4096


def _gate_block(x_ref, w_ref, gates_ref, idx_ref):
    logits = lax.dot_general(
        w_ref[...], x_ref[...],
        dimension_numbers=(((1,), (1,)), ((), ())),
        preferred_element_type=jnp.float32,
    )
    row = lax.broadcasted_iota(jnp.int32, logits.shape, 0)
    neg_big = jnp.finfo(jnp.float32).min

    m1 = jnp.max(logits, axis=0)
    i1 = jnp.min(jnp.where(logits == m1[None, :], row, NUM_EXPERTS), axis=0)
    masked = jnp.where(row == i1[None, :], neg_big, logits)
    m2 = jnp.max(masked, axis=0)
    i2 = jnp.min(jnp.where(masked == m2[None, :], row, NUM_EXPERTS), axis=0)

    e = jnp.exp(m2 - m1)
    g1 = 1.0 / (1.0 + e)
    g2 = 1.0 - g1
    gates_ref[...] = jnp.stack([g1, g2], axis=0)
    idx_ref[...] = jnp.stack([i1, i2], axis=0)


@jax.jit
def kernel(x, gate_weight):
    n = x.shape[0]
    grid = (n // BLOCK_T,)
    gates_t, idx_t = pl.pallas_call(
        _gate_block,
        grid=grid,
        in_specs=[
            pl.BlockSpec((BLOCK_T, HIDDEN_SIZE), lambda i: (i, 0)),
            pl.BlockSpec((NUM_EXPERTS, HIDDEN_SIZE), lambda i: (0, 0)),
        ],
        out_specs=[
            pl.BlockSpec((2, BLOCK_T), lambda i: (0, i)),
            pl.BlockSpec((2, BLOCK_T), lambda i: (0, i)),
        ],
        out_shape=[
            jax.ShapeDtypeStruct((2, n), jnp.float32),
            jax.ShapeDtypeStruct((2, n), jnp.int32),
        ],
    )(x, gate_weight)
    return (gates_t.T, idx_t.T)
